# Initial kernel scaffold; baseline (speedup 1.0000x reference)
import jax
import jax.numpy as jnp
from jax.experimental import pallas as pl


def kernel(node_features, edge_index, proj_W, proj_b, W1, b1, W2, b2):
    raise NotImplementedError("write your pallas kernel here")


# speedup vs baseline: 11.6079x; 11.6079x over previous
import functools

import jax
import jax.numpy as jnp
from jax import lax
from jax.experimental import pallas as pl
from jax.experimental.pallas import tpu as pltpu
from jax.experimental.pallas import tpu_sc as plsc

_NC, _NS, _L = 2, 16, 16
_NW = _NC * _NS
_CH = 80
_ZCH = 16


def _row_partition(N):
    main = (N // (_NS * 8)) * 8
    last = N - main * (_NS - 1)
    assert main % _ZCH == 0 and last % _ZCH == 0 and last >= main
    return main, main // _ZCH, last // _ZCH


def _each_span(sid, N, fn):
    main, nmain, nlast = _row_partition(N)
    base = sid * main

    def body(j, carry):
        fn(base + j * _ZCH)
        return carry

    lax.fori_loop(0, nmain, body, None)

    @pl.when(sid == _NS - 1)
    def _():
        def body2(j, carry):
            fn(base + j * _ZCH)
            return carry
        lax.fori_loop(nmain, nlast, body2, None)


def _make_deg_kernel(E, N):
    ept = E // _NW
    nchunk = ept // _CH
    assert E == ept * _NW and ept == nchunk * _CH
    mesh = plsc.VectorSubcoreMesh(core_axis_name="c", subcore_axis_name="s")

    @functools.partial(
        pl.kernel,
        out_type=jax.ShapeDtypeStruct((_NC, N, _L), jnp.float32),
        mesh=mesh,
        scratch_types=[
            pltpu.VMEM_SHARED((N, _L), jnp.float32),
            pltpu.VMEM((_CH, _L), jnp.float32),
            pltpu.VMEM((_ZCH, _L), jnp.float32),
            pltpu.VMEM((_CH,), jnp.int32),
        ],
    )
    def deg_kernel(dst_hbm, out_hbm, acc, ones_v, zero_v, idx_v):
        cid = lax.axis_index("c")
        sid = lax.axis_index("s")

        def fill(i, carry):
            @pl.when(i < _CH)
            def _():
                ones_v[i, :] = jnp.ones((_L,), jnp.float32)

            @pl.when(i < _ZCH)
            def _():
                zero_v[i, :] = jnp.zeros((_L,), jnp.float32)
            return carry

        lax.fori_loop(0, max(_CH, _ZCH), fill, None)

        _each_span(sid, N,
                   lambda r: pltpu.sync_copy(zero_v, acc.at[pl.ds(r, _ZCH)]))
        plsc.subcore_barrier()

        base = (cid * _NS + sid) * ept

        def chunk(i, carry):
            pltpu.sync_copy(dst_hbm.at[pl.ds(base + i * _CH, _CH)], idx_v)
            pltpu.sync_copy(ones_v, acc.at[idx_v], add=True)
            return carry

        lax.fori_loop(0, nchunk, chunk, None)
        plsc.subcore_barrier()
        _each_span(sid, N,
                   lambda r: pltpu.sync_copy(acc.at[pl.ds(r, _ZCH)],
                                             out_hbm.at[cid, pl.ds(r, _ZCH)]))

    return deg_kernel


def _make_mp_kernel(E, N, D):
    ept = E // _NW
    nchunk = ept // _CH
    assert E == ept * _NW and ept == nchunk * _CH and D % _L == 0
    mesh = plsc.VectorSubcoreMesh(core_axis_name="c", subcore_axis_name="s")

    @functools.partial(
        pl.kernel,
        out_type=jax.ShapeDtypeStruct((_NC, N, D), jnp.float32),
        mesh=mesh,
        scratch_types=[
            pltpu.VMEM_SHARED((N, D), jnp.float32),
            pltpu.VMEM((_ZCH, D), jnp.float32),
            pltpu.VMEM((_CH, D), jnp.float32),
            pltpu.VMEM((_CH,), jnp.int32),
            pltpu.VMEM((_CH,), jnp.int32),
            pltpu.SemaphoreType.DMA,
        ],
    )
    def mp_kernel(y_hbm, src_hbm, dst_hbm, out_hbm,
                  acc, zero_v, rows_v, src_v, dst_v, sem):
        cid = lax.axis_index("c")
        sid = lax.axis_index("s")

        def fill(i, carry):
            r = i // (D // _L)
            c = i % (D // _L)
            zero_v[r, pl.ds(c * _L, _L)] = jnp.zeros((_L,), jnp.float32)
            return carry

        lax.fori_loop(0, _ZCH * (D // _L), fill, None)

        _each_span(sid, N,
                   lambda r: pltpu.sync_copy(zero_v, acc.at[pl.ds(r, _ZCH)]))
        plsc.subcore_barrier()

        base = (cid * _NS + sid) * ept

        def chunk(i, carry):
            pltpu.sync_copy(src_hbm.at[pl.ds(base + i * _CH, _CH)], src_v)
            pltpu.sync_copy(dst_hbm.at[pl.ds(base + i * _CH, _CH)], dst_v)
            pltpu.async_copy(y_hbm.at[src_v], rows_v, sem).wait()
            pltpu.sync_copy(rows_v, acc.at[dst_v], add=True)
            return carry

        lax.fori_loop(0, nchunk, chunk, None)
        plsc.subcore_barrier()
        _each_span(sid, N,
                   lambda r: pltpu.sync_copy(acc.at[pl.ds(r, _ZCH)],
                                             out_hbm.at[cid, pl.ds(r, _ZCH)]))

    return mp_kernel


_RB = 400


def _dinv_block(degpair_ref):
    deg = degpair_ref[0, :, 0:1] + degpair_ref[1, :, 0:1] + 1.0
    return lax.rsqrt(deg)


def _tc1_body(degpair_ref, nf_ref, pw_ref, pb_ref, w1_ref, y1_ref):
    dinv = _dinv_block(degpair_ref)
    x0 = jnp.dot(nf_ref[...], pw_ref[...],
                 preferred_element_type=jnp.float32,
                 precision=lax.Precision.HIGHEST) + pb_ref[...]
    xw = jnp.dot(x0, w1_ref[...], preferred_element_type=jnp.float32,
                 precision=lax.Precision.HIGHEST)
    y1_ref[...] = xw * dinv


def _tc2_body(degpair_ref, accp_ref, y1_ref, b1_ref, w2_ref, y2_ref):
    dinv = _dinv_block(degpair_ref)
    acc = accp_ref[0] + accp_ref[1]
    h = jnp.maximum(dinv * (acc + y1_ref[...]) + b1_ref[...], 0.0)
    xw = jnp.dot(h, w2_ref[...], preferred_element_type=jnp.float32,
                 precision=lax.Precision.HIGHEST)
    y2_ref[...] = xw * dinv


def _tc3_body(degpair_ref, accp_ref, y2_ref, b2_ref, out_ref):
    dinv = _dinv_block(degpair_ref)
    acc = accp_ref[0] + accp_ref[1]
    out_ref[...] = jnp.maximum(dinv * (acc + y2_ref[...]) + b2_ref[...], 0.0)


def _row_spec(d):
    return pl.BlockSpec((_RB, d), lambda i: (i, 0))


def _pair_spec(d):
    return pl.BlockSpec((_NC, _RB, d), lambda i: (0, i, 0))


def _full_spec(r, c):
    return pl.BlockSpec((r, c), lambda i: (0, 0))


def kernel(node_features, edge_index, proj_W, proj_b, W1, b1, W2, b2):
    N, in_dim = node_features.shape
    E = edge_index.shape[1]
    proj_dim = proj_W.shape[1]
    hid = W1.shape[1]

    ei = edge_index.astype(jnp.int32)
    src, dst = ei[0], ei[1]

    degpair = _make_deg_kernel(E, N)(dst)

    grid = (N // _RB,)
    y1 = pl.pallas_call(
        _tc1_body,
        grid=grid,
        in_specs=[_pair_spec(_L), _row_spec(in_dim),
                  _full_spec(in_dim, proj_dim), _full_spec(1, proj_dim),
                  _full_spec(proj_dim, hid)],
        out_specs=_row_spec(hid),
        out_shape=jax.ShapeDtypeStruct((N, hid), jnp.float32),
    )(degpair, node_features, proj_W, proj_b.reshape(1, -1), W1)

    mp = _make_mp_kernel(E, N, hid)
    accp1 = mp(y1, src, dst)

    y2 = pl.pallas_call(
        _tc2_body,
        grid=grid,
        in_specs=[_pair_spec(_L), _pair_spec(hid), _row_spec(hid),
                  _full_spec(1, hid), _full_spec(hid, hid)],
        out_specs=_row_spec(hid),
        out_shape=jax.ShapeDtypeStruct((N, hid), jnp.float32),
    )(degpair, accp1, y1, b1.reshape(1, -1), W2)

    accp2 = mp(y2, src, dst)

    out = pl.pallas_call(
        _tc3_body,
        grid=grid,
        in_specs=[_pair_spec(_L), _pair_spec(hid), _row_spec(hid),
                  _full_spec(1, hid)],
        out_specs=_row_spec(hid),
        out_shape=jax.ShapeDtypeStruct((N, hid), jnp.float32),
    )(degpair, accp2, y2, b2.reshape(1, -1))

    return out


# speedup vs baseline: 25.4198x; 2.1899x over previous
import functools

import jax
import jax.numpy as jnp
from jax import lax
from jax.experimental import pallas as pl
from jax.experimental.pallas import tpu as pltpu
from jax.experimental.pallas import tpu_sc as plsc

_NC, _NS, _L = 2, 16, 16
_NW = _NC * _NS
_CH = 125
_NBUF = 4


def _span_copy(sid, N, copy_fn):
    main = (N // (_NS * 8)) * 8
    last = N - main * (_NS - 1)

    @pl.when(sid < _NS - 1)
    def _():
        copy_fn(sid * main, main)

    @pl.when(sid == _NS - 1)
    def _():
        copy_fn((_NS - 1) * main, last)


def _make_deg_kernel(E, N):
    ept = E // _NW
    nchunk = ept // _CH
    assert E == ept * _NW and ept == nchunk * _CH
    mesh = plsc.VectorSubcoreMesh(core_axis_name="c", subcore_axis_name="s")

    @functools.partial(
        pl.kernel,
        out_type=jax.ShapeDtypeStruct((_NC, N, _L), jnp.float32),
        mesh=mesh,
        compiler_params=pltpu.CompilerParams(use_tc_tiling_on_sc=False),
        scratch_types=[
            pltpu.VMEM_SHARED((N, _L), jnp.float32),
            pltpu.VMEM((_CH, _L), jnp.float32),
            pltpu.VMEM((nchunk, _CH), jnp.int32),
            pltpu.SemaphoreType.DMA,
        ],
    )
    def deg_kernel(dst3_hbm, zero_hbm, out_hbm, acc, ones_v, idx_v, sem):
        cid = lax.axis_index("c")
        sid = lax.axis_index("s")
        wid = cid * _NS + sid

        def fill(i, carry):
            ones_v[i, :] = jnp.ones((_L,), jnp.float32)
            return carry

        lax.fori_loop(0, _CH, fill, None)
        pltpu.sync_copy(dst3_hbm.at[wid], idx_v)
        _span_copy(sid, N,
                   lambda r, n: pltpu.sync_copy(zero_hbm.at[pl.ds(r, n)],
                                                acc.at[pl.ds(r, n)]))
        plsc.subcore_barrier()

        def fire(i, carry):
            pltpu.async_copy(ones_v, acc.at[idx_v.at[i]], sem, add=True)
            return carry

        lax.fori_loop(0, nchunk, fire, None)

        def drain(i, carry):
            pltpu.make_async_copy(ones_v, acc.at[idx_v.at[0]], sem).wait()
            return carry

        lax.fori_loop(0, nchunk, drain, None)
        plsc.subcore_barrier()
        _span_copy(sid, N,
                   lambda r, n: pltpu.sync_copy(acc.at[pl.ds(r, n)],
                                                out_hbm.at[cid, pl.ds(r, n)]))

    return deg_kernel


def _make_mp_kernel(E, N, D):
    dh = D // _NC
    ept = E // _NS
    nchunk = ept // _CH
    assert E == ept * _NS and ept == nchunk * _CH and D == dh * _NC
    assert nchunk % _NBUF == 0 and nchunk // _NBUF >= 2
    nmain = nchunk // _NBUF - 1
    mesh = plsc.VectorSubcoreMesh(core_axis_name="c", subcore_axis_name="s")

    @functools.partial(
        pl.kernel,
        out_type=jax.ShapeDtypeStruct((_NC, N, dh), jnp.float32),
        mesh=mesh,
        compiler_params=pltpu.CompilerParams(use_tc_tiling_on_sc=False),
        scratch_types=[
            pltpu.VMEM_SHARED((N, dh), jnp.float32),
            pltpu.VMEM((_NBUF, _CH, dh), jnp.float32),
            pltpu.VMEM((nchunk, _CH), jnp.int32),
            pltpu.VMEM((nchunk, _CH), jnp.int32),
            [pltpu.SemaphoreType.DMA] * _NBUF,
            [pltpu.SemaphoreType.DMA] * _NBUF,
        ],
    )
    def mp_kernel(y3_hbm, src3_hbm, dst3_hbm, zero_hbm, out_hbm,
                  acc, rows_v, src_v, dst_v, gsem, ssem):
        cid = lax.axis_index("c")
        sid = lax.axis_index("s")
        yh = y3_hbm.at[cid]

        pltpu.sync_copy(src3_hbm.at[sid], src_v)
        pltpu.sync_copy(dst3_hbm.at[sid], dst_v)
        _span_copy(sid, N,
                   lambda r, n: pltpu.sync_copy(zero_hbm.at[pl.ds(r, n)],
                                                acc.at[pl.ds(r, n)]))
        plsc.subcore_barrier()

        def fire_gather(g, b):
            pltpu.async_copy(yh.at[src_v.at[g]], rows_v.at[b], gsem[b])

        def wait_gather(b):
            pltpu.make_async_copy(yh.at[src_v.at[0]], rows_v.at[b],
                                  gsem[b]).wait()

        def fire_scatter(g, b):
            pltpu.async_copy(rows_v.at[b], acc.at[dst_v.at[g]], ssem[b],
                             add=True)

        def wait_scatter(b):
            pltpu.make_async_copy(rows_v.at[b], acc.at[dst_v.at[0]],
                                  ssem[b]).wait()

        for b in range(_NBUF):
            fire_gather(b, b)

        def ring(k, carry):
            g0 = k * _NBUF
            for b in range(_NBUF):
                wait_gather(b)
                fire_scatter(g0 + b, b)
            for b in range(_NBUF):
                wait_scatter(b)
                fire_gather(g0 + _NBUF + b, b)
            return carry

        lax.fori_loop(0, nmain, ring, None)

        g0 = nmain * _NBUF
        for b in range(_NBUF):
            wait_gather(b)
            fire_scatter(g0 + b, b)
        for b in range(_NBUF):
            wait_scatter(b)

        plsc.subcore_barrier()
        _span_copy(sid, N,
                   lambda r, n: pltpu.sync_copy(acc.at[pl.ds(r, n)],
                                                out_hbm.at[cid, pl.ds(r, n)]))

    return mp_kernel


_RB = 400


def _dinv_block(degpair_ref):
    deg = degpair_ref[0, :, 0:1] + degpair_ref[1, :, 0:1] + 1.0
    return lax.rsqrt(deg)


def _halves(pair_ref):
    return jnp.concatenate([pair_ref[0], pair_ref[1]], axis=1)


def _store_halves(pair_ref, x, dh):
    pair_ref[0] = x[:, :dh]
    pair_ref[1] = x[:, dh:]


def _tc1_body(degpair_ref, nf_ref, pw_ref, pb_ref, w1_ref, y1_ref):
    dinv = _dinv_block(degpair_ref)
    x0 = jnp.dot(nf_ref[...], pw_ref[...],
                 preferred_element_type=jnp.float32,
                 precision=lax.Precision.HIGHEST) + pb_ref[...]
    xw = jnp.dot(x0, w1_ref[...], preferred_element_type=jnp.float32,
                 precision=lax.Precision.HIGHEST)
    _store_halves(y1_ref, xw * dinv, w1_ref.shape[1] // _NC)


def _tc2_body(degpair_ref, accp_ref, y1_ref, b1_ref, w2_ref, y2_ref):
    dinv = _dinv_block(degpair_ref)
    h = jnp.maximum(dinv * (_halves(accp_ref) + _halves(y1_ref))
                    + b1_ref[...], 0.0)
    xw = jnp.dot(h, w2_ref[...], preferred_element_type=jnp.float32,
                 precision=lax.Precision.HIGHEST)
    _store_halves(y2_ref, xw * dinv, w2_ref.shape[1] // _NC)


def _tc3_body(degpair_ref, accp_ref, y2_ref, b2_ref, out_ref):
    dinv = _dinv_block(degpair_ref)
    out_ref[...] = jnp.maximum(
        dinv * (_halves(accp_ref) + _halves(y2_ref)) + b2_ref[...], 0.0)


def _row_spec(d):
    return pl.BlockSpec((_RB, d), lambda i: (i, 0))


def _pair_spec(d):
    return pl.BlockSpec((_NC, _RB, d), lambda i: (0, i, 0))


def _full_spec(r, c):
    return pl.BlockSpec((r, c), lambda i: (0, 0))


def kernel(node_features, edge_index, proj_W, proj_b, W1, b1, W2, b2):
    N, in_dim = node_features.shape
    E = edge_index.shape[1]
    proj_dim = proj_W.shape[1]
    hid = W1.shape[1]
    dh = hid // _NC
    nchunk_deg = E // (_NW * _CH)
    nchunk_mp = E // (_NS * _CH)

    ei = edge_index.astype(jnp.int32)
    dst3_deg = ei[1].reshape(_NW, nchunk_deg, _CH)
    src3 = ei[0].reshape(_NS, nchunk_mp, _CH)
    dst3 = ei[1].reshape(_NS, nchunk_mp, _CH)
    zeros16 = jnp.zeros((N, _L), jnp.float32)
    zeros_dh = jnp.zeros((N, dh), jnp.float32)

    degpair = _make_deg_kernel(E, N)(dst3_deg, zeros16)

    grid = (N // _RB,)
    y1 = pl.pallas_call(
        _tc1_body,
        grid=grid,
        in_specs=[_pair_spec(_L), _row_spec(in_dim),
                  _full_spec(in_dim, proj_dim), _full_spec(1, proj_dim),
                  _full_spec(proj_dim, hid)],
        out_specs=_pair_spec(dh),
        out_shape=jax.ShapeDtypeStruct((_NC, N, dh), jnp.float32),
    )(degpair, node_features, proj_W, proj_b.reshape(1, -1), W1)

    mp = _make_mp_kernel(E, N, hid)
    accp1 = mp(y1, src3, dst3, zeros_dh)

    y2 = pl.pallas_call(
        _tc2_body,
        grid=grid,
        in_specs=[_pair_spec(_L), _pair_spec(dh), _pair_spec(dh),
                  _full_spec(1, hid), _full_spec(hid, hid)],
        out_specs=_pair_spec(dh),
        out_shape=jax.ShapeDtypeStruct((_NC, N, dh), jnp.float32),
    )(degpair, accp1, y1, b1.reshape(1, -1), W2)

    accp2 = mp(y2, src3, dst3, zeros_dh)

    out = pl.pallas_call(
        _tc3_body,
        grid=grid,
        in_specs=[_pair_spec(_L), _pair_spec(dh), _pair_spec(dh),
                  _full_spec(1, hid)],
        out_specs=_row_spec(hid),
        out_shape=jax.ShapeDtypeStruct((N, hid), jnp.float32),
    )(degpair, accp2, y2, b2.reshape(1, -1))

    return out


# speedup vs baseline: 30.0674x; 1.1828x over previous
import functools

import jax
import jax.numpy as jnp
from jax import lax
from jax.experimental import pallas as pl
from jax.experimental.pallas import tpu as pltpu
from jax.experimental.pallas import tpu_sc as plsc

_NC, _NS, _L = 2, 16, 16
_NW = _NC * _NS
_CH = 125
_NBUF = 5


def _span_copy(sid, N, copy_fn):
    main = (N // (_NS * 8)) * 8
    last = N - main * (_NS - 1)

    @pl.when(sid < _NS - 1)
    def _():
        copy_fn(sid * main, main)

    @pl.when(sid == _NS - 1)
    def _():
        copy_fn((_NS - 1) * main, last)


def _make_deg_kernel(E, N):
    ept = E // _NW
    nchunk = ept // _CH
    assert E == ept * _NW and ept == nchunk * _CH
    mesh = plsc.VectorSubcoreMesh(core_axis_name="c", subcore_axis_name="s")

    @functools.partial(
        pl.kernel,
        out_type=jax.ShapeDtypeStruct((_NC, N, _L), jnp.float32),
        mesh=mesh,
        compiler_params=pltpu.CompilerParams(use_tc_tiling_on_sc=False),
        scratch_types=[
            pltpu.VMEM_SHARED((N, _L), jnp.float32),
            pltpu.VMEM((_CH, _L), jnp.float32),
            pltpu.VMEM((nchunk, _CH), jnp.int32),
            pltpu.SemaphoreType.DMA,
        ],
    )
    def deg_kernel(ei4_hbm, zero_hbm, out_hbm, acc, ones_v, idx_v, sem):
        cid = lax.axis_index("c")
        sid = lax.axis_index("s")
        wid = cid * _NS + sid

        def fill(i, carry):
            ones_v[i, :] = jnp.ones((_L,), jnp.float32)
            return carry

        lax.fori_loop(0, _CH, fill, None)
        pltpu.sync_copy(ei4_hbm.at[1, wid], idx_v)
        _span_copy(sid, N,
                   lambda r, n: pltpu.sync_copy(zero_hbm.at[pl.ds(r, n)],
                                                acc.at[pl.ds(r, n)]))
        plsc.subcore_barrier()

        def fire(i, carry):
            pltpu.async_copy(ones_v, acc.at[idx_v.at[i]], sem, add=True)
            return carry

        lax.fori_loop(0, nchunk, fire, None)

        def drain(i, carry):
            pltpu.make_async_copy(ones_v, acc.at[idx_v.at[0]], sem).wait()
            return carry

        lax.fori_loop(0, nchunk, drain, None)
        plsc.subcore_barrier()
        _span_copy(sid, N,
                   lambda r, n: pltpu.sync_copy(acc.at[pl.ds(r, n)],
                                                out_hbm.at[cid, pl.ds(r, n)]))

    return deg_kernel


def _make_mp_kernel(E, N, D):
    dh = D // _NC
    ept = E // _NS
    nchunk = ept // _CH
    assert E == ept * _NS and ept == nchunk * _CH and D == dh * _NC
    assert nchunk % _NBUF == 0 and nchunk // _NBUF >= 2
    nmain = nchunk // _NBUF - 1
    mesh = plsc.VectorSubcoreMesh(core_axis_name="c", subcore_axis_name="s")

    @functools.partial(
        pl.kernel,
        out_type=jax.ShapeDtypeStruct((_NC, N, dh), jnp.float32),
        mesh=mesh,
        compiler_params=pltpu.CompilerParams(use_tc_tiling_on_sc=False),
        scratch_types=[
            pltpu.VMEM_SHARED((N, dh), jnp.float32),
            pltpu.VMEM((_NBUF, _CH, dh), jnp.float32),
            pltpu.VMEM((nchunk, _CH), jnp.int32),
            pltpu.VMEM((nchunk, _CH), jnp.int32),
            [pltpu.SemaphoreType.DMA] * _NBUF,
            [pltpu.SemaphoreType.DMA] * _NBUF,
        ],
    )
    def mp_kernel(y3_hbm, ei4_hbm, zero_hbm, out_hbm,
                  acc, rows_v, src_v, dst_v, gsem, ssem):
        cid = lax.axis_index("c")
        sid = lax.axis_index("s")
        yh = y3_hbm.at[cid]

        pltpu.sync_copy(ei4_hbm.at[0, sid], src_v)
        pltpu.sync_copy(ei4_hbm.at[1, sid], dst_v)
        _span_copy(sid, N,
                   lambda r, n: pltpu.sync_copy(zero_hbm.at[pl.ds(r, n)],
                                                acc.at[pl.ds(r, n)]))
        plsc.subcore_barrier()

        def fire_gather(g, b):
            pltpu.async_copy(yh.at[src_v.at[g]], rows_v.at[b], gsem[b])

        def wait_gather(b):
            pltpu.make_async_copy(yh.at[src_v.at[0]], rows_v.at[b],
                                  gsem[b]).wait()

        def fire_scatter(g, b):
            pltpu.async_copy(rows_v.at[b], acc.at[dst_v.at[g]], ssem[b],
                             add=True)

        def wait_scatter(b):
            pltpu.make_async_copy(rows_v.at[b], acc.at[dst_v.at[0]],
                                  ssem[b]).wait()

        for b in range(_NBUF):
            fire_gather(b, b)

        def ring(k, carry):
            g0 = k * _NBUF
            for b in range(_NBUF):
                wait_gather(b)
                fire_scatter(g0 + b, b)
            for b in range(_NBUF):
                wait_scatter(b)
                fire_gather(g0 + _NBUF + b, b)
            return carry

        lax.fori_loop(0, nmain, ring, None)

        g0 = nmain * _NBUF
        for b in range(_NBUF):
            wait_gather(b)
            fire_scatter(g0 + b, b)
        for b in range(_NBUF):
            wait_scatter(b)

        plsc.subcore_barrier()
        _span_copy(sid, N,
                   lambda r, n: pltpu.sync_copy(acc.at[pl.ds(r, n)],
                                                out_hbm.at[cid, pl.ds(r, n)]))

    return mp_kernel


_RB = 2000


def _dinv_block(degpair_ref):
    deg = degpair_ref[0, :, 0:1] + degpair_ref[1, :, 0:1] + 1.0
    return lax.rsqrt(deg)


def _halves(pair_ref):
    return jnp.concatenate([pair_ref[0], pair_ref[1]], axis=1)


def _store_halves(pair_ref, x, dh):
    pair_ref[0] = x[:, :dh]
    pair_ref[1] = x[:, dh:]


def _tc1_body(degpair_ref, nf_ref, pw_ref, pb_ref, w1_ref, y1_ref):
    dinv = _dinv_block(degpair_ref)
    x0 = jnp.dot(nf_ref[...], pw_ref[...],
                 preferred_element_type=jnp.float32) + pb_ref[...]
    xw = jnp.dot(x0, w1_ref[...], preferred_element_type=jnp.float32)
    _store_halves(y1_ref, xw * dinv, w1_ref.shape[1] // _NC)


def _tc2_body(degpair_ref, accp_ref, y1_ref, b1_ref, w2_ref, y2_ref):
    dinv = _dinv_block(degpair_ref)
    h = jnp.maximum(dinv * (_halves(accp_ref) + _halves(y1_ref))
                    + b1_ref[...], 0.0)
    xw = jnp.dot(h, w2_ref[...], preferred_element_type=jnp.float32)
    _store_halves(y2_ref, xw * dinv, w2_ref.shape[1] // _NC)


def _tc3_body(degpair_ref, accp_ref, y2_ref, b2_ref, out_ref):
    dinv = _dinv_block(degpair_ref)
    out_ref[...] = jnp.maximum(
        dinv * (_halves(accp_ref) + _halves(y2_ref)) + b2_ref[...], 0.0)


def _row_spec(d):
    return pl.BlockSpec((_RB, d), lambda i: (i, 0))


def _pair_spec(d):
    return pl.BlockSpec((_NC, _RB, d), lambda i: (0, i, 0))


def _full_spec(r, c):
    return pl.BlockSpec((r, c), lambda i: (0, 0))


def kernel(node_features, edge_index, proj_W, proj_b, W1, b1, W2, b2):
    N, in_dim = node_features.shape
    E = edge_index.shape[1]
    proj_dim = proj_W.shape[1]
    hid = W1.shape[1]
    dh = hid // _NC
    nchunk_deg = E // (_NW * _CH)
    nchunk_mp = E // (_NS * _CH)

    ei = edge_index.astype(jnp.int32)
    ei4_deg = ei.reshape(2, _NW, nchunk_deg, _CH)
    ei4_mp = ei.reshape(2, _NS, nchunk_mp, _CH)
    zeros16 = jnp.zeros((N, _L), jnp.float32)
    zeros_dh = jnp.zeros((N, dh), jnp.float32)

    degpair = _make_deg_kernel(E, N)(ei4_deg, zeros16)

    grid = (N // _RB,)
    y1 = pl.pallas_call(
        _tc1_body,
        grid=grid,
        in_specs=[_pair_spec(_L), _row_spec(in_dim),
                  _full_spec(in_dim, proj_dim), _full_spec(1, proj_dim),
                  _full_spec(proj_dim, hid)],
        out_specs=_pair_spec(dh),
        out_shape=jax.ShapeDtypeStruct((_NC, N, dh), jnp.float32),
    )(degpair, node_features, proj_W, proj_b.reshape(1, -1), W1)

    mp = _make_mp_kernel(E, N, hid)
    accp1 = mp(y1, ei4_mp, zeros_dh)

    y2 = pl.pallas_call(
        _tc2_body,
        grid=grid,
        in_specs=[_pair_spec(_L), _pair_spec(dh), _pair_spec(dh),
                  _full_spec(1, hid), _full_spec(hid, hid)],
        out_specs=_pair_spec(dh),
        out_shape=jax.ShapeDtypeStruct((_NC, N, dh), jnp.float32),
    )(degpair, accp1, y1, b1.reshape(1, -1), W2)

    accp2 = mp(y2, ei4_mp, zeros_dh)

    out = pl.pallas_call(
        _tc3_body,
        grid=grid,
        in_specs=[_pair_spec(_L), _pair_spec(dh), _pair_spec(dh),
                  _full_spec(1, hid)],
        out_specs=_row_spec(hid),
        out_shape=jax.ShapeDtypeStruct((N, hid), jnp.float32),
    )(degpair, accp2, y2, b2.reshape(1, -1))

    return out
